# baseline (speedup 1.0000x reference)
import functools

import jax
import jax.numpy as jnp
from jax import lax
from jax.experimental import pallas as pl
from jax.experimental.pallas import tpu as pltpu
from jax.experimental.pallas import tpu_sc as plsc

NC = 2
NS = 16
CHUNK = 128
NBUF = 4
LAG = 2
LANES = 16


def _mesh():
    return plsc.VectorSubcoreMesh(core_axis_name="c", subcore_axis_name="s")


@functools.cache
def _deg_sc(n_pad, ch):
    rpt = n_pad // NS
    ch2 = ch // 2

    @functools.partial(
        pl.kernel,
        out_type=jax.ShapeDtypeStruct((NC, n_pad, LANES), jnp.float32),
        mesh=_mesh(),
        compiler_params=pltpu.CompilerParams(use_tc_tiling_on_sc=False),
        scratch_types=[
            pltpu.VMEM_SHARED((n_pad, LANES), jnp.float32),
            pltpu.VMEM((ch, CHUNK), jnp.int32),
            pltpu.VMEM((CHUNK, LANES), jnp.float32),
        ],
    )
    def k(ei_hbm, z_hbm, ones_hbm, out_hbm, acc_sh, idx_v, ones_v):
        cid = lax.axis_index("c")
        tid = lax.axis_index("s")
        r0 = tid * rpt
        pltpu.sync_copy(ones_hbm, ones_v)
        pltpu.sync_copy(ei_hbm.at[0, tid], idx_v)
        pltpu.sync_copy(z_hbm.at[pl.ds(r0, rpt)], acc_sh.at[pl.ds(r0, rpt)])
        plsc.subcore_barrier()

        def step(j, carry):
            pltpu.sync_copy(ones_v, acc_sh.at[idx_v.at[j]], add=True)
            return carry

        lax.fori_loop(cid * ch2, cid * ch2 + ch2, step, 0)
        plsc.subcore_barrier()
        pltpu.sync_copy(acc_sh.at[pl.ds(r0, rpt)],
                        out_hbm.at[cid, pl.ds(r0, rpt)])

    return k


@functools.cache
def _seg_sum_sc(n_pad, hid, ch64):
    rpt = n_pad // NS
    chn = ch64 // NC
    C2 = 64

    @functools.partial(
        pl.kernel,
        out_type=jax.ShapeDtypeStruct((NC, n_pad, hid), jnp.float32),
        mesh=_mesh(),
        compiler_params=pltpu.CompilerParams(use_tc_tiling_on_sc=False),
        scratch_types=[
            pltpu.VMEM_SHARED((n_pad, hid), jnp.float32),
            pltpu.VMEM((chn, C2), jnp.int32),
            pltpu.VMEM((chn, C2), jnp.int32),
            [pltpu.VMEM((C2, hid), jnp.float32) for _ in range(2)],
            [pltpu.SemaphoreType.DMA for _ in range(2)],
        ],
    )
    def k(g_hbm, ei_hbm, z_hbm, out_hbm, acc_sh, src_v, dst_v, rows, sems):
        cid = lax.axis_index("c")
        tid = lax.axis_index("s")
        r0 = tid * rpt
        jb = cid * chn
        pltpu.sync_copy(ei_hbm.at[0, tid, pl.ds(jb, chn)], src_v)
        pltpu.sync_copy(ei_hbm.at[1, tid, pl.ds(jb, chn)], dst_v)
        pltpu.sync_copy(z_hbm.at[pl.ds(r0, rpt)], acc_sh.at[pl.ds(r0, rpt)])
        plsc.subcore_barrier()

        for b in range(2):
            pltpu.async_copy(g_hbm.at[src_v.at[b]], rows[b], sems[b])

        def step(i, carry):
            j0 = i * 2
            for b in range(2):
                j = j0 + b
                pltpu.make_async_copy(g_hbm.at[src_v.at[0]], rows[b],
                                      sems[b]).wait()
                pltpu.sync_copy(rows[b], acc_sh.at[dst_v.at[j]], add=True)

                @pl.when(j + 2 < chn)
                def _():
                    pltpu.async_copy(g_hbm.at[src_v.at[j + 2]], rows[b],
                                     sems[b])

            return carry

        lax.fori_loop(0, chn // 2, step, 0)
        plsc.subcore_barrier()
        pltpu.sync_copy(acc_sh.at[pl.ds(r0, rpt)],
                        out_hbm.at[cid, pl.ds(r0, rpt)])

    return k


def _dis_from(deg_rep):
    deg = deg_rep[0, :, 0] + deg_rep[1, :, 0]
    safe = jnp.where(deg > 0, deg, 1.0)
    return jnp.where(deg > 0, lax.rsqrt(safe), 0.0)


def _mm1_body(x_ref, w_ref, b_ref, deg_ref, h_ref, g_ref):
    dis = _dis_from(deg_ref[...])
    h = jnp.maximum(jnp.dot(x_ref[...], w_ref[...],
                            preferred_element_type=jnp.float32) + b_ref[...], 0.0)
    h_ref[...] = h
    g_ref[...] = h * dis[:, None]


def _mm2_body(h_ref, acc_ref, deg_ref, w20_ref, w21_ref, b2_ref,
              wl_ref, bl_ref, o_ref):
    dis = _dis_from(deg_ref[...])
    tx1 = (acc_ref[0] + acc_ref[1]) * (-dis)[:, None]
    h = h_ref[...]
    t = jnp.dot(h, w20_ref[...], preferred_element_type=jnp.float32)
    t = t + jnp.dot(tx1, w21_ref[...], preferred_element_type=jnp.float32)
    t = jnp.maximum(t + b2_ref[...], 0.0) + h
    logits = jnp.dot(t, wl_ref[...], preferred_element_type=jnp.float32) + bl_ref[...]
    m = jnp.max(logits, axis=1, keepdims=True)
    s = logits - m
    lse = jnp.log(jnp.sum(jnp.exp(s), axis=1, keepdims=True))
    o_ref[...] = s - lse


def kernel(x, edge_index, W1, b1, W2, b2, Wl, bl):
    n, f_in = x.shape
    e = edge_index.shape[1]
    hid = W2.shape[-1]
    ncls = Wl.shape[-1]

    grain = NS * CHUNK * NBUF
    e_pad = -(-e // grain) * grain
    ch = e_pad // (NS * CHUNK)
    n_pad = -(-n // (NS * 64)) * (NS * 64)

    ei_flat = jnp.concatenate(
        [edge_index.astype(jnp.int32),
         jnp.full((2, e_pad - e), n, jnp.int32)], axis=1)
    ei = ei_flat.reshape(2, NS, ch, CHUNK)
    ch64 = e_pad // (NS * 64)
    ei64 = ei_flat.reshape(2, NS, ch64, 64)
    x_pad = jnp.zeros((n_pad, f_in), jnp.float32).at[:n].set(x)

    z16 = jnp.zeros((n_pad, LANES), jnp.float32)
    ones = jnp.ones((CHUNK, LANES), jnp.float32)
    zh = jnp.zeros((n_pad, hid), jnp.float32)

    deg_rep = _deg_sc(n_pad, ch)(ei, z16, ones)

    blk1 = 640
    g1 = n_pad // blk1
    h_pad, g_pad = pl.pallas_call(
        _mm1_body,
        grid=(g1,),
        in_specs=[
            pl.BlockSpec((blk1, f_in), lambda i: (i, 0)),
            pl.BlockSpec((f_in, hid), lambda i: (0, 0)),
            pl.BlockSpec((1, hid), lambda i: (0, 0)),
            pl.BlockSpec((NC, blk1, LANES), lambda i: (0, i, 0)),
        ],
        out_specs=[
            pl.BlockSpec((blk1, hid), lambda i: (i, 0)),
            pl.BlockSpec((blk1, hid), lambda i: (i, 0)),
        ],
        out_shape=[
            jax.ShapeDtypeStruct((n_pad, hid), jnp.float32),
            jax.ShapeDtypeStruct((n_pad, hid), jnp.float32),
        ],
    )(x_pad, W1[0], b1.reshape(1, hid), deg_rep)

    acc = _seg_sum_sc(n_pad, hid, ch64)(g_pad, ei64, zh)

    blk2 = 1000
    g2 = n // blk2
    out = pl.pallas_call(
        _mm2_body,
        grid=(g2,),
        in_specs=[
            pl.BlockSpec((blk2, hid), lambda i: (i, 0)),
            pl.BlockSpec((NC, blk2, hid), lambda i: (0, i, 0)),
            pl.BlockSpec((NC, blk2, LANES), lambda i: (0, i, 0)),
            pl.BlockSpec((hid, hid), lambda i: (0, 0)),
            pl.BlockSpec((hid, hid), lambda i: (0, 0)),
            pl.BlockSpec((1, hid), lambda i: (0, 0)),
            pl.BlockSpec((hid, ncls), lambda i: (0, 0)),
            pl.BlockSpec((1, ncls), lambda i: (0, 0)),
        ],
        out_specs=pl.BlockSpec((blk2, ncls), lambda i: (i, 0)),
        out_shape=jax.ShapeDtypeStruct((n, ncls), jnp.float32),
    )(h_pad, acc, deg_rep, W2[0], W2[1], b2.reshape(1, hid),
      Wl, bl.reshape(1, ncls))

    return out


# speedup vs baseline: 1.9193x; 1.9193x over previous
import functools

import jax
import jax.numpy as jnp
from jax import lax
from jax.experimental import pallas as pl
from jax.experimental.pallas import tpu as pltpu
from jax.experimental.pallas import tpu_sc as plsc

NC = 2
NS = 16
CHUNK = 128
NBUF = 2
WCH = 40
LANES = 16


def _mesh():
    return plsc.VectorSubcoreMesh(core_axis_name="c", subcore_axis_name="s")


@functools.cache
def _deg_sc(n_pad, ch):
    rpt = n_pad // NS
    ch2 = ch // 2

    @functools.partial(
        pl.kernel,
        out_type=jax.ShapeDtypeStruct((NC, n_pad, LANES), jnp.float32),
        mesh=_mesh(),
        compiler_params=pltpu.CompilerParams(use_tc_tiling_on_sc=False),
        scratch_types=[
            pltpu.VMEM_SHARED((n_pad, LANES), jnp.float32),
            pltpu.VMEM((ch, CHUNK), jnp.int32),
            pltpu.VMEM((CHUNK, LANES), jnp.float32),
        ],
    )
    def k(ei_hbm, z_hbm, ones_hbm, out_hbm, acc_sh, idx_v, ones_v):
        cid = lax.axis_index("c")
        tid = lax.axis_index("s")
        r0 = tid * rpt
        pltpu.sync_copy(ones_hbm, ones_v)
        pltpu.sync_copy(ei_hbm.at[0, tid], idx_v)
        pltpu.sync_copy(z_hbm.at[pl.ds(r0, rpt)], acc_sh.at[pl.ds(r0, rpt)])
        plsc.subcore_barrier()

        def step(j, carry):
            pltpu.sync_copy(ones_v, acc_sh.at[idx_v.at[j]], add=True)
            return carry

        lax.fori_loop(cid * ch2, cid * ch2 + ch2, step, 0)
        plsc.subcore_barrier()
        pltpu.sync_copy(acc_sh.at[pl.ds(r0, rpt)],
                        out_hbm.at[cid, pl.ds(r0, rpt)])

    return k


@functools.cache
def _seg_sum_sc(n_pad, half, ch):
    rpt = n_pad // NS
    W = WCH
    assert ch % W == 0

    @functools.partial(
        pl.kernel,
        out_type=jax.ShapeDtypeStruct((NC, n_pad, half), jnp.float32),
        mesh=_mesh(),
        compiler_params=pltpu.CompilerParams(use_tc_tiling_on_sc=False),
        scratch_types=[
            pltpu.VMEM_SHARED((n_pad, half), jnp.float32),
            pltpu.VMEM_SHARED((n_pad, half), jnp.float32),
            pltpu.VMEM((W, CHUNK), jnp.int32),
            pltpu.VMEM((W, CHUNK), jnp.int32),
            [pltpu.VMEM((CHUNK, half), jnp.float32) for _ in range(NBUF)],
            [pltpu.SemaphoreType.DMA for _ in range(NBUF)],
        ],
    )
    def k(g_hbm, ei_hbm, z_hbm, out_hbm, acc_sh, g_sh, src_v, dst_v,
          rows, gsems):
        cid = lax.axis_index("c")
        tid = lax.axis_index("s")
        r0 = tid * rpt
        pltpu.sync_copy(g_hbm.at[cid, pl.ds(r0, rpt)], g_sh.at[pl.ds(r0, rpt)])
        pltpu.sync_copy(z_hbm.at[pl.ds(r0, rpt)], acc_sh.at[pl.ds(r0, rpt)])
        plsc.subcore_barrier()

        def wloop(w, carry):
            pltpu.sync_copy(ei_hbm.at[0, tid, pl.ds(w * W, W)], src_v)
            pltpu.sync_copy(ei_hbm.at[1, tid, pl.ds(w * W, W)], dst_v)
            for b in range(NBUF):
                pltpu.async_copy(g_sh.at[src_v.at[b]], rows[b], gsems[b])

            def step(i, carry2):
                j0 = i * NBUF
                for b in range(NBUF):
                    j = j0 + b
                    pltpu.make_async_copy(g_sh.at[src_v.at[0]], rows[b],
                                          gsems[b]).wait()
                    pltpu.sync_copy(rows[b], acc_sh.at[dst_v.at[j]], add=True)

                    @pl.when(j + NBUF < W)
                    def _():
                        pltpu.async_copy(g_sh.at[src_v.at[j + NBUF]], rows[b],
                                         gsems[b])

                return carry2

            lax.fori_loop(0, W // NBUF, step, 0)
            return carry

        lax.fori_loop(0, ch // W, wloop, 0)
        plsc.subcore_barrier()
        pltpu.sync_copy(acc_sh.at[pl.ds(r0, rpt)],
                        out_hbm.at[cid, pl.ds(r0, rpt)])

    return k


def _dis_from(deg_rep):
    deg = deg_rep[0, :, 0] + deg_rep[1, :, 0]
    safe = jnp.where(deg > 0, deg, 1.0)
    return jnp.where(deg > 0, lax.rsqrt(safe), 0.0)


def _mm1_body(x_ref, w_ref, b_ref, deg_ref, h_ref, g_ref):
    dis = _dis_from(deg_ref[...])
    h = jnp.maximum(jnp.dot(x_ref[...], w_ref[...],
                            preferred_element_type=jnp.float32) + b_ref[...], 0.0)
    h_ref[...] = h
    g = h * dis[:, None]
    half = g.shape[1] // 2
    g_ref[0] = g[:, :half]
    g_ref[1] = g[:, half:]


def _mm2_body(h_ref, acc_ref, deg_ref, w20_ref, w21_ref, b2_ref,
              wl_ref, bl_ref, o_ref):
    dis = _dis_from(deg_ref[...])
    tx1 = jnp.concatenate([acc_ref[0], acc_ref[1]], axis=1) * (-dis)[:, None]
    h = h_ref[...]
    t = jnp.dot(h, w20_ref[...], preferred_element_type=jnp.float32)
    t = t + jnp.dot(tx1, w21_ref[...], preferred_element_type=jnp.float32)
    t = jnp.maximum(t + b2_ref[...], 0.0) + h
    logits = jnp.dot(t, wl_ref[...], preferred_element_type=jnp.float32) + bl_ref[...]
    m = jnp.max(logits, axis=1, keepdims=True)
    s = logits - m
    lse = jnp.log(jnp.sum(jnp.exp(s), axis=1, keepdims=True))
    o_ref[...] = s - lse


def kernel(x, edge_index, W1, b1, W2, b2, Wl, bl):
    n, f_in = x.shape
    e = edge_index.shape[1]
    hid = W2.shape[-1]
    half = hid // 2
    ncls = Wl.shape[-1]

    grain = NS * CHUNK * WCH
    e_pad = -(-e // grain) * grain
    ch = e_pad // (NS * CHUNK)
    n_pad = -(-n // (NS * 64)) * (NS * 64)

    ei = jnp.concatenate(
        [edge_index.astype(jnp.int32),
         jnp.full((2, e_pad - e), n, jnp.int32)], axis=1
    ).reshape(2, NS, ch, CHUNK)
    x_pad = jnp.zeros((n_pad, f_in), jnp.float32).at[:n].set(x)

    z16 = jnp.zeros((n_pad, LANES), jnp.float32)
    ones = jnp.ones((CHUNK, LANES), jnp.float32)
    zh = jnp.zeros((n_pad, half), jnp.float32)

    deg_rep = _deg_sc(n_pad, ch)(ei, z16, ones)

    blk1 = 640
    g1 = n_pad // blk1
    h_pad, g_split = pl.pallas_call(
        _mm1_body,
        grid=(g1,),
        in_specs=[
            pl.BlockSpec((blk1, f_in), lambda i: (i, 0)),
            pl.BlockSpec((f_in, hid), lambda i: (0, 0)),
            pl.BlockSpec((1, hid), lambda i: (0, 0)),
            pl.BlockSpec((NC, blk1, LANES), lambda i: (0, i, 0)),
        ],
        out_specs=[
            pl.BlockSpec((blk1, hid), lambda i: (i, 0)),
            pl.BlockSpec((NC, blk1, half), lambda i: (0, i, 0)),
        ],
        out_shape=[
            jax.ShapeDtypeStruct((n_pad, hid), jnp.float32),
            jax.ShapeDtypeStruct((NC, n_pad, half), jnp.float32),
        ],
    )(x_pad, W1[0], b1.reshape(1, hid), deg_rep)

    acc = _seg_sum_sc(n_pad, half, ch)(g_split, ei, zh)

    blk2 = 1000
    g2 = n // blk2
    out = pl.pallas_call(
        _mm2_body,
        grid=(g2,),
        in_specs=[
            pl.BlockSpec((blk2, hid), lambda i: (i, 0)),
            pl.BlockSpec((NC, blk2, half), lambda i: (0, i, 0)),
            pl.BlockSpec((NC, blk2, LANES), lambda i: (0, i, 0)),
            pl.BlockSpec((hid, hid), lambda i: (0, 0)),
            pl.BlockSpec((hid, hid), lambda i: (0, 0)),
            pl.BlockSpec((1, hid), lambda i: (0, 0)),
            pl.BlockSpec((hid, ncls), lambda i: (0, 0)),
            pl.BlockSpec((1, ncls), lambda i: (0, 0)),
        ],
        out_specs=pl.BlockSpec((blk2, ncls), lambda i: (i, 0)),
        out_shape=jax.ShapeDtypeStruct((n, ncls), jnp.float32),
    )(h_pad, acc, deg_rep, W2[0], W2[1], b2.reshape(1, hid),
      Wl, bl.reshape(1, ncls))

    return out


# speedup vs baseline: 2.0797x; 1.0835x over previous
import functools

import jax
import jax.numpy as jnp
from jax import lax
from jax.experimental import pallas as pl
from jax.experimental.pallas import tpu as pltpu
from jax.experimental.pallas import tpu_sc as plsc

NC = 2
NS = 16
CHUNK = 128
NBUF = 5
LAG = 2
WCH = 20
LANES = 16


def _mesh():
    return plsc.VectorSubcoreMesh(core_axis_name="c", subcore_axis_name="s")


@functools.cache
def _deg_sc(n_pad, ch):
    rpt = n_pad // NS
    ch2 = ch // 2

    @functools.partial(
        pl.kernel,
        out_type=jax.ShapeDtypeStruct((NC, n_pad, LANES), jnp.float32),
        mesh=_mesh(),
        compiler_params=pltpu.CompilerParams(use_tc_tiling_on_sc=False),
        scratch_types=[
            pltpu.VMEM_SHARED((n_pad, LANES), jnp.float32),
            pltpu.VMEM((ch, CHUNK), jnp.int32),
            pltpu.VMEM((CHUNK, LANES), jnp.float32),
        ],
    )
    def k(ei_hbm, z_hbm, ones_hbm, out_hbm, acc_sh, idx_v, ones_v):
        cid = lax.axis_index("c")
        tid = lax.axis_index("s")
        r0 = tid * rpt
        pltpu.sync_copy(ones_hbm, ones_v)
        pltpu.sync_copy(ei_hbm.at[0, tid], idx_v)
        pltpu.sync_copy(z_hbm.at[pl.ds(r0, rpt)], acc_sh.at[pl.ds(r0, rpt)])
        plsc.subcore_barrier()

        def step(j, carry):
            pltpu.sync_copy(ones_v, acc_sh.at[idx_v.at[j]], add=True)
            return carry

        lax.fori_loop(cid * ch2, cid * ch2 + ch2, step, 0)
        plsc.subcore_barrier()
        pltpu.sync_copy(acc_sh.at[pl.ds(r0, rpt)],
                        out_hbm.at[cid, pl.ds(r0, rpt)])

    return k


@functools.cache
def _seg_sum_sc(n_pad, half, ch):
    rpt = n_pad // NS
    W = WCH
    assert ch % W == 0

    @functools.partial(
        pl.kernel,
        out_type=jax.ShapeDtypeStruct((NC, n_pad, half), jnp.float32),
        mesh=_mesh(),
        compiler_params=pltpu.CompilerParams(use_tc_tiling_on_sc=False),
        scratch_types=[
            pltpu.VMEM_SHARED((n_pad, half), jnp.float32),
            pltpu.VMEM_SHARED((n_pad, half), jnp.float32),
            pltpu.VMEM((W, CHUNK), jnp.int32),
            pltpu.VMEM((W, CHUNK), jnp.int32),
            [pltpu.VMEM((CHUNK, half), jnp.float32) for _ in range(NBUF)],
            [pltpu.SemaphoreType.DMA for _ in range(NBUF)],
            [pltpu.SemaphoreType.DMA for _ in range(NBUF)],
        ],
    )
    def k(g_hbm, ei_hbm, z_hbm, out_hbm, acc_sh, g_sh, src_v, dst_v,
          rows, gsems, ssems):
        cid = lax.axis_index("c")
        tid = lax.axis_index("s")
        r0 = tid * rpt
        pltpu.sync_copy(g_hbm.at[cid, pl.ds(r0, rpt)], g_sh.at[pl.ds(r0, rpt)])
        pltpu.sync_copy(z_hbm.at[pl.ds(r0, rpt)], acc_sh.at[pl.ds(r0, rpt)])
        plsc.subcore_barrier()

        def wloop(w, carry):
            pltpu.sync_copy(ei_hbm.at[0, tid, pl.ds(w * W, W)], src_v)
            pltpu.sync_copy(ei_hbm.at[1, tid, pl.ds(w * W, W)], dst_v)
            for b in range(NBUF):
                pltpu.async_copy(g_sh.at[src_v.at[b]], rows[b], gsems[b])

            def step(i, carry2):
                j0 = i * NBUF
                for b in range(NBUF):
                    j = j0 + b
                    pltpu.make_async_copy(g_sh.at[src_v.at[0]], rows[b],
                                          gsems[b]).wait()
                    pltpu.async_copy(rows[b], acc_sh.at[dst_v.at[j]],
                                     ssems[b], add=True)
                    bk = (b - LAG) % NBUF
                    k_ = j - LAG

                    @pl.when(k_ >= 0)
                    def _():
                        pltpu.make_async_copy(rows[bk],
                                              acc_sh.at[dst_v.at[0]],
                                              ssems[bk]).wait()

                        @pl.when(k_ + NBUF < W)
                        def _():
                            pltpu.async_copy(g_sh.at[src_v.at[k_ + NBUF]],
                                             rows[bk], gsems[bk])

                return carry2

            lax.fori_loop(0, W // NBUF, step, 0)
            for b in range(LAG):
                bb = (W - LAG + b) % NBUF
                pltpu.make_async_copy(rows[bb], acc_sh.at[dst_v.at[0]],
                                      ssems[bb]).wait()
            return carry

        lax.fori_loop(0, ch // W, wloop, 0)
        plsc.subcore_barrier()
        pltpu.sync_copy(acc_sh.at[pl.ds(r0, rpt)],
                        out_hbm.at[cid, pl.ds(r0, rpt)])

    return k


def _dis_from(deg_rep):
    deg = deg_rep[0, :, 0] + deg_rep[1, :, 0]
    safe = jnp.where(deg > 0, deg, 1.0)
    return jnp.where(deg > 0, lax.rsqrt(safe), 0.0)


def _mm1_body(x_ref, w_ref, b_ref, deg_ref, h_ref, g_ref):
    dis = _dis_from(deg_ref[...])
    h = jnp.maximum(jnp.dot(x_ref[...], w_ref[...],
                            preferred_element_type=jnp.float32) + b_ref[...], 0.0)
    h_ref[...] = h
    g = h * dis[:, None]
    half = g.shape[1] // 2
    g_ref[0] = g[:, :half]
    g_ref[1] = g[:, half:]


def _mm2_body(h_ref, acc_ref, deg_ref, w20_ref, w21_ref, b2_ref,
              wl_ref, bl_ref, o_ref):
    dis = _dis_from(deg_ref[...])
    tx1 = jnp.concatenate([acc_ref[0], acc_ref[1]], axis=1) * (-dis)[:, None]
    h = h_ref[...]
    t = jnp.dot(h, w20_ref[...], preferred_element_type=jnp.float32)
    t = t + jnp.dot(tx1, w21_ref[...], preferred_element_type=jnp.float32)
    t = jnp.maximum(t + b2_ref[...], 0.0) + h
    logits = jnp.dot(t, wl_ref[...], preferred_element_type=jnp.float32) + bl_ref[...]
    m = jnp.max(logits, axis=1, keepdims=True)
    s = logits - m
    lse = jnp.log(jnp.sum(jnp.exp(s), axis=1, keepdims=True))
    o_ref[...] = s - lse


def kernel(x, edge_index, W1, b1, W2, b2, Wl, bl):
    n, f_in = x.shape
    e = edge_index.shape[1]
    hid = W2.shape[-1]
    half = hid // 2
    ncls = Wl.shape[-1]

    grain = NS * CHUNK * WCH
    e_pad = -(-e // grain) * grain
    ch = e_pad // (NS * CHUNK)
    n_pad = -(-n // (NS * 64)) * (NS * 64)

    ei = jnp.concatenate(
        [edge_index.astype(jnp.int32),
         jnp.full((2, e_pad - e), n, jnp.int32)], axis=1
    ).reshape(2, NS, ch, CHUNK)
    x_pad = jnp.zeros((n_pad, f_in), jnp.float32).at[:n].set(x)

    z16 = jnp.zeros((n_pad, LANES), jnp.float32)
    ones = jnp.ones((CHUNK, LANES), jnp.float32)
    zh = jnp.zeros((n_pad, half), jnp.float32)

    deg_rep = _deg_sc(n_pad, ch)(ei, z16, ones)

    blk1 = 640
    g1 = n_pad // blk1
    h_pad, g_split = pl.pallas_call(
        _mm1_body,
        grid=(g1,),
        in_specs=[
            pl.BlockSpec((blk1, f_in), lambda i: (i, 0)),
            pl.BlockSpec((f_in, hid), lambda i: (0, 0)),
            pl.BlockSpec((1, hid), lambda i: (0, 0)),
            pl.BlockSpec((NC, blk1, LANES), lambda i: (0, i, 0)),
        ],
        out_specs=[
            pl.BlockSpec((blk1, hid), lambda i: (i, 0)),
            pl.BlockSpec((NC, blk1, half), lambda i: (0, i, 0)),
        ],
        out_shape=[
            jax.ShapeDtypeStruct((n_pad, hid), jnp.float32),
            jax.ShapeDtypeStruct((NC, n_pad, half), jnp.float32),
        ],
    )(x_pad, W1[0], b1.reshape(1, hid), deg_rep)

    acc = _seg_sum_sc(n_pad, half, ch)(g_split, ei, zh)

    blk2 = 1000
    g2 = n // blk2
    out = pl.pallas_call(
        _mm2_body,
        grid=(g2,),
        in_specs=[
            pl.BlockSpec((blk2, hid), lambda i: (i, 0)),
            pl.BlockSpec((NC, blk2, half), lambda i: (0, i, 0)),
            pl.BlockSpec((NC, blk2, LANES), lambda i: (0, i, 0)),
            pl.BlockSpec((hid, hid), lambda i: (0, 0)),
            pl.BlockSpec((hid, hid), lambda i: (0, 0)),
            pl.BlockSpec((1, hid), lambda i: (0, 0)),
            pl.BlockSpec((hid, ncls), lambda i: (0, 0)),
            pl.BlockSpec((1, ncls), lambda i: (0, 0)),
        ],
        out_specs=pl.BlockSpec((blk2, ncls), lambda i: (i, 0)),
        out_shape=jax.ShapeDtypeStruct((n, ncls), jnp.float32),
    )(h_pad, acc, deg_rep, W2[0], W2[1], b2.reshape(1, hid),
      Wl, bl.reshape(1, ncls))

    return out


# speedup vs baseline: 2.1550x; 1.0362x over previous
import functools

import jax
import jax.numpy as jnp
from jax import lax
from jax.experimental import pallas as pl
from jax.experimental.pallas import tpu as pltpu
from jax.experimental.pallas import tpu_sc as plsc

NC = 2
NS = 16
CHUNK = 128
NBUF = 4
LAG = 2
WCH = 40
LANES = 16


def _mesh():
    return plsc.VectorSubcoreMesh(core_axis_name="c", subcore_axis_name="s")


@functools.cache
def _deg_sc(n_pad, ch):
    rpt = n_pad // NS
    ch2 = ch // 2

    @functools.partial(
        pl.kernel,
        out_type=jax.ShapeDtypeStruct((NC, n_pad, LANES), jnp.float32),
        mesh=_mesh(),
        compiler_params=pltpu.CompilerParams(use_tc_tiling_on_sc=False),
        scratch_types=[
            pltpu.VMEM_SHARED((n_pad, LANES), jnp.float32),
            pltpu.VMEM((ch, CHUNK), jnp.int32),
            pltpu.VMEM((CHUNK, LANES), jnp.float32),
        ],
    )
    def k(ei_hbm, z_hbm, ones_hbm, out_hbm, acc_sh, idx_v, ones_v):
        cid = lax.axis_index("c")
        tid = lax.axis_index("s")
        r0 = tid * rpt
        pltpu.sync_copy(ones_hbm, ones_v)
        pltpu.sync_copy(ei_hbm.at[0, tid], idx_v)
        pltpu.sync_copy(z_hbm.at[pl.ds(r0, rpt)], acc_sh.at[pl.ds(r0, rpt)])
        plsc.subcore_barrier()

        def step(j, carry):
            pltpu.sync_copy(ones_v, acc_sh.at[idx_v.at[j]], add=True)
            return carry

        lax.fori_loop(cid * ch2, cid * ch2 + ch2, step, 0)
        plsc.subcore_barrier()
        pltpu.sync_copy(acc_sh.at[pl.ds(r0, rpt)],
                        out_hbm.at[cid, pl.ds(r0, rpt)])

    return k


@functools.cache
def _seg_sum_sc(n_pad, half, ch):
    rpt = n_pad // NS
    W = WCH
    assert ch % W == 0

    @functools.partial(
        pl.kernel,
        out_type=jax.ShapeDtypeStruct((NC, n_pad, half), jnp.float32),
        mesh=_mesh(),
        compiler_params=pltpu.CompilerParams(use_tc_tiling_on_sc=False),
        scratch_types=[
            pltpu.VMEM_SHARED((n_pad, half), jnp.float32),
            pltpu.VMEM_SHARED((n_pad, half), jnp.float32),
            pltpu.VMEM((W, CHUNK), jnp.int32),
            pltpu.VMEM((W, CHUNK), jnp.int32),
            [pltpu.VMEM((CHUNK, half), jnp.float32) for _ in range(NBUF)],
            [pltpu.SemaphoreType.DMA for _ in range(NBUF)],
            [pltpu.SemaphoreType.DMA for _ in range(NBUF)],
        ],
    )
    def k(g_hbm, ei_hbm, z_hbm, out_hbm, acc_sh, g_sh, src_v, dst_v,
          rows, gsems, ssems):
        cid = lax.axis_index("c")
        tid = lax.axis_index("s")
        r0 = tid * rpt
        pltpu.sync_copy(g_hbm.at[cid, pl.ds(r0, rpt)], g_sh.at[pl.ds(r0, rpt)])
        pltpu.sync_copy(z_hbm.at[pl.ds(r0, rpt)], acc_sh.at[pl.ds(r0, rpt)])
        plsc.subcore_barrier()

        def wloop(w, carry):
            pltpu.sync_copy(ei_hbm.at[0, tid, pl.ds(w * W, W)], src_v)
            pltpu.sync_copy(ei_hbm.at[1, tid, pl.ds(w * W, W)], dst_v)
            for b in range(NBUF):
                pltpu.async_copy(g_sh.at[src_v.at[b]], rows[b], gsems[b])

            def step(i, carry2):
                j0 = i * NBUF
                for b in range(NBUF):
                    j = j0 + b
                    pltpu.make_async_copy(g_sh.at[src_v.at[0]], rows[b],
                                          gsems[b]).wait()
                    pltpu.async_copy(rows[b], acc_sh.at[dst_v.at[j]],
                                     ssems[b], add=True)
                    bk = (b - LAG) % NBUF
                    k_ = j - LAG

                    @pl.when(k_ >= 0)
                    def _():
                        pltpu.make_async_copy(rows[bk],
                                              acc_sh.at[dst_v.at[0]],
                                              ssems[bk]).wait()

                        @pl.when(k_ + NBUF < W)
                        def _():
                            pltpu.async_copy(g_sh.at[src_v.at[k_ + NBUF]],
                                             rows[bk], gsems[bk])

                return carry2

            lax.fori_loop(0, W // NBUF, step, 0)
            for b in range(LAG):
                bb = (W - LAG + b) % NBUF
                pltpu.make_async_copy(rows[bb], acc_sh.at[dst_v.at[0]],
                                      ssems[bb]).wait()
            return carry

        lax.fori_loop(0, ch // W, wloop, 0)
        plsc.subcore_barrier()
        pltpu.sync_copy(acc_sh.at[pl.ds(r0, rpt)],
                        out_hbm.at[cid, pl.ds(r0, rpt)])

    return k


def _dis_from(deg_rep):
    deg = deg_rep[0, :, 0] + deg_rep[1, :, 0]
    safe = jnp.where(deg > 0, deg, 1.0)
    return jnp.where(deg > 0, lax.rsqrt(safe), 0.0)


def _mm1a_body(x_ref, w_ref, b_ref, h_ref):
    h_ref[...] = jnp.maximum(
        jnp.dot(x_ref[...], w_ref[...],
                preferred_element_type=jnp.float32) + b_ref[...], 0.0)


def _mm1b_body(h_ref, deg_ref, g_ref):
    dis = _dis_from(deg_ref[...])
    g = h_ref[...] * dis[:, None]
    half = g.shape[1] // 2
    g_ref[0] = g[:, :half]
    g_ref[1] = g[:, half:]


def _mm2_body(h_ref, acc_ref, deg_ref, w20_ref, w21_ref, b2_ref,
              wl_ref, bl_ref, o_ref):
    dis = _dis_from(deg_ref[...])
    tx1 = jnp.concatenate([acc_ref[0], acc_ref[1]], axis=1) * (-dis)[:, None]
    h = h_ref[...]
    t = jnp.dot(h, w20_ref[...], preferred_element_type=jnp.float32)
    t = t + jnp.dot(tx1, w21_ref[...], preferred_element_type=jnp.float32)
    t = jnp.maximum(t + b2_ref[...], 0.0) + h
    logits = jnp.dot(t, wl_ref[...], preferred_element_type=jnp.float32) + bl_ref[...]
    m = jnp.max(logits, axis=1, keepdims=True)
    s = logits - m
    lse = jnp.log(jnp.sum(jnp.exp(s), axis=1, keepdims=True))
    o_ref[...] = s - lse


def kernel(x, edge_index, W1, b1, W2, b2, Wl, bl):
    n, f_in = x.shape
    e = edge_index.shape[1]
    hid = W2.shape[-1]
    half = hid // 2
    ncls = Wl.shape[-1]

    grain = NS * CHUNK * WCH
    e_pad = -(-e // grain) * grain
    ch = e_pad // (NS * CHUNK)
    n_pad = -(-n // (NS * 64)) * (NS * 64)

    ei = jnp.concatenate(
        [edge_index.astype(jnp.int32),
         jnp.full((2, e_pad - e), n, jnp.int32)], axis=1
    ).reshape(2, NS, ch, CHUNK)
    x_pad = jnp.zeros((n_pad, f_in), jnp.float32).at[:n].set(x)

    z16 = jnp.zeros((n_pad, LANES), jnp.float32)
    ones = jnp.ones((CHUNK, LANES), jnp.float32)
    zh = jnp.zeros((n_pad, half), jnp.float32)

    deg_rep = _deg_sc(n_pad, ch)(ei, z16, ones)

    blk1 = 640
    g1 = n_pad // blk1
    h_pad = pl.pallas_call(
        _mm1a_body,
        grid=(g1,),
        in_specs=[
            pl.BlockSpec((blk1, f_in), lambda i: (i, 0)),
            pl.BlockSpec((f_in, hid), lambda i: (0, 0)),
            pl.BlockSpec((1, hid), lambda i: (0, 0)),
        ],
        out_specs=pl.BlockSpec((blk1, hid), lambda i: (i, 0)),
        out_shape=jax.ShapeDtypeStruct((n_pad, hid), jnp.float32),
    )(x_pad, W1[0], b1.reshape(1, hid))
    g_split = pl.pallas_call(
        _mm1b_body,
        grid=(g1,),
        in_specs=[
            pl.BlockSpec((blk1, hid), lambda i: (i, 0)),
            pl.BlockSpec((NC, blk1, LANES), lambda i: (0, i, 0)),
        ],
        out_specs=pl.BlockSpec((NC, blk1, half), lambda i: (0, i, 0)),
        out_shape=jax.ShapeDtypeStruct((NC, n_pad, half), jnp.float32),
    )(h_pad, deg_rep)

    acc = _seg_sum_sc(n_pad, half, ch)(g_split, ei, zh)

    blk2 = 1000
    g2 = n // blk2
    out = pl.pallas_call(
        _mm2_body,
        grid=(g2,),
        in_specs=[
            pl.BlockSpec((blk2, hid), lambda i: (i, 0)),
            pl.BlockSpec((NC, blk2, half), lambda i: (0, i, 0)),
            pl.BlockSpec((NC, blk2, LANES), lambda i: (0, i, 0)),
            pl.BlockSpec((hid, hid), lambda i: (0, 0)),
            pl.BlockSpec((hid, hid), lambda i: (0, 0)),
            pl.BlockSpec((1, hid), lambda i: (0, 0)),
            pl.BlockSpec((hid, ncls), lambda i: (0, 0)),
            pl.BlockSpec((1, ncls), lambda i: (0, 0)),
        ],
        out_specs=pl.BlockSpec((blk2, ncls), lambda i: (i, 0)),
        out_shape=jax.ShapeDtypeStruct((n, ncls), jnp.float32),
    )(h_pad, acc, deg_rep, W2[0], W2[1], b2.reshape(1, hid),
      Wl, bl.reshape(1, ncls))

    return out
